# baseline (speedup 1.0000x reference)
import functools

import jax
import jax.numpy as jnp
from jax.experimental import pallas as pl
from jax.experimental.pallas import tpu as pltpu


def _round_up(n, m):
    return ((n + m - 1) // m) * m


def _logz_partial_kernel(x_ref, wt_ref, b_ref, lzp_ref, m_sc, l_sc, *,
                         batch, tile_b, tiles_per_core, masked):
    c = pl.program_id(0)
    i = pl.program_id(1)

    @pl.when(i == 0)
    def _():
        m_sc[...] = jnp.full_like(m_sc, -jnp.inf)
        l_sc[...] = jnp.zeros_like(l_sc)

    h = jnp.dot(x_ref[...], wt_ref[...],
                preferred_element_type=jnp.float32) + b_ref[...]

    if masked:
        row = (c * tiles_per_core + i) * tile_b + jax.lax.broadcasted_iota(
            jnp.int32, h.shape, 0)
        h = jnp.where(row < batch, h, -jnp.inf)

    m_prev = m_sc[...]
    m_new = jnp.maximum(m_prev, jnp.max(h, axis=0, keepdims=True))
    p = jnp.exp(h - m_new)
    l_sc[...] = l_sc[...] * jnp.exp(m_prev - m_new) + jnp.sum(
        p, axis=0, keepdims=True)
    m_sc[...] = m_new

    @pl.when(i == tiles_per_core - 1)
    def _():
        lzp_ref[...] = (m_sc[...] + jnp.log(l_sc[...]))[None]


def _normalize_kernel(x_ref, wt_ref, b_ref, lzp_ref, out_ref):
    lz = lzp_ref[...]
    m = jnp.max(lz, axis=0)
    logz = m + jnp.log(jnp.sum(jnp.exp(lz - m), axis=0))
    h = jnp.dot(x_ref[...], wt_ref[...],
                preferred_element_type=jnp.float32) + b_ref[...]
    out_ref[...] = h - logz


def kernel(x, w1, b1, w2, b2, w3, b3):
    B, F = x.shape
    assert F == 10

    w_eff = (w3 @ w2 @ w1).astype(jnp.float32)
    b_eff = (w3 @ (w2 @ b1 + b2) + b3).astype(jnp.float32)
    wt = w_eff.T
    brow = b_eff.reshape(1, 6)

    tile_b = 8192
    b_pad = _round_up(B, tile_b)
    n_tiles = b_pad // tile_b
    n_cores = 2 if n_tiles >= 2 else 1
    n_tiles = _round_up(n_tiles, n_cores)
    b_pad = n_tiles * tile_b
    tiles_per_core = n_tiles // n_cores
    masked = b_pad != B

    xp = x
    if masked:
        xp = jnp.zeros((b_pad, F), x.dtype).at[:B].set(x)

    full = lambda *_: (0, 0)
    w_spec = pl.BlockSpec((10, 6), full)
    b_spec = pl.BlockSpec((1, 6), full)

    lz_part = pl.pallas_call(
        functools.partial(_logz_partial_kernel, batch=B, tile_b=tile_b,
                          tiles_per_core=tiles_per_core, masked=masked),
        out_shape=jax.ShapeDtypeStruct((n_cores, 1, 6), jnp.float32),
        grid_spec=pltpu.PrefetchScalarGridSpec(
            num_scalar_prefetch=0,
            grid=(n_cores, tiles_per_core),
            in_specs=[
                pl.BlockSpec((tile_b, 10),
                             lambda c, i, tpc=tiles_per_core: (c * tpc + i, 0)),
                pl.BlockSpec((10, 6), lambda c, i: (0, 0)),
                pl.BlockSpec((1, 6), lambda c, i: (0, 0)),
            ],
            out_specs=pl.BlockSpec((1, 1, 6), lambda c, i: (c, 0, 0)),
            scratch_shapes=[pltpu.VMEM((1, 6), jnp.float32),
                            pltpu.VMEM((1, 6), jnp.float32)],
        ),
        compiler_params=pltpu.CompilerParams(
            dimension_semantics=("parallel", "arbitrary")),
        cost_estimate=pl.CostEstimate(
            flops=2 * b_pad * 10 * 6,
            transcendentals=6 * b_pad,
            bytes_accessed=(b_pad * 10 + 10 * 6 + 6 + 2 * 6) * 4,
        ),
    )(xp, wt, brow)

    out = pl.pallas_call(
        _normalize_kernel,
        out_shape=jax.ShapeDtypeStruct((b_pad, 6), jnp.float32),
        grid_spec=pltpu.PrefetchScalarGridSpec(
            num_scalar_prefetch=0,
            grid=(n_tiles,),
            in_specs=[
                pl.BlockSpec((tile_b, 10), lambda i: (i, 0)),
                pl.BlockSpec((10, 6), lambda i: (0, 0)),
                pl.BlockSpec((1, 6), lambda i: (0, 0)),
                pl.BlockSpec((n_cores, 1, 6), lambda i: (0, 0, 0)),
            ],
            out_specs=pl.BlockSpec((tile_b, 6), lambda i: (i, 0)),
        ),
        compiler_params=pltpu.CompilerParams(
            dimension_semantics=("parallel",)),
        cost_estimate=pl.CostEstimate(
            flops=2 * b_pad * 10 * 6,
            transcendentals=2 * 6 * n_tiles,
            bytes_accessed=(b_pad * 10 + b_pad * 6 + 10 * 6 + 6) * 4,
        ),
    )(xp, wt, brow, lz_part)

    if masked:
        out = out[:B]
    return out
